# baseline (speedup 1.0000x reference)
import functools

import jax
import jax.numpy as jnp
from jax import lax
from jax.experimental import pallas as pl
from jax.experimental.pallas import tpu as pltpu
from jax.experimental.pallas import tpu_sc as plsc

NC = 2
NS = 16
NW = NC * NS
BE = 128
BN = 1024


def _mesh():
    return plsc.VectorSubcoreMesh(core_axis_name="c", subcore_axis_name="s")


DW = 16


@functools.lru_cache(maxsize=None)
def _degree_kernel(NP, KE):
    rpt = NP // NS

    def body(dst3, zrows, ones_h, out, acc, didx, ones_v):
        c = lax.axis_index("c")
        s = lax.axis_index("s")
        r0 = s * rpt
        pltpu.sync_copy(dst3.at[c, s], didx)
        pltpu.sync_copy(zrows, acc.at[pl.ds(r0, rpt)])
        pltpu.sync_copy(ones_h, ones_v)
        plsc.subcore_barrier()

        @pl.loop(0, KE)
        def _scat(j):
            pltpu.sync_copy(ones_v, acc.at[didx.at[j]], add=True)

        plsc.subcore_barrier()
        pltpu.sync_copy(acc.at[pl.ds(r0, rpt)], out.at[c, pl.ds(r0, rpt)])

    return pl.kernel(
        body,
        out_type=jax.ShapeDtypeStruct((NC, NP, DW), jnp.float32),
        mesh=_mesh(),
        scratch_types=[
            pltpu.VMEM_SHARED((NP, DW), jnp.float32),
            pltpu.VMEM((KE, BE), jnp.int32),
            pltpu.VMEM((BE, DW), jnp.float32),
        ],
        compiler_params=pltpu.CompilerParams(use_tc_tiling_on_sc=False),
    )


@functools.lru_cache(maxsize=None)
def _propagate_kernel(NP, KE, D):
    rpt = NP // NS

    KE2 = KE // 2

    def body(table, src3, dst3, zrows, out, acc, sidx, didx, rows, sems):
        c = lax.axis_index("c")
        s = lax.axis_index("s")
        r0 = s * rpt
        pltpu.sync_copy(zrows, acc.at[pl.ds(r0, rpt)])
        plsc.subcore_barrier()

        for p in range(2):
            pltpu.sync_copy(src3.at[c, s, pl.ds(p * KE2, KE2)], sidx)
            pltpu.sync_copy(dst3.at[c, s, pl.ds(p * KE2, KE2)], didx)
            pltpu.async_copy(table.at[sidx.at[0]], rows.at[0], sems.at[0])
            pltpu.async_copy(table.at[sidx.at[1]], rows.at[1], sems.at[1])

            @pl.loop(0, KE2 // 2)
            def _go(g):
                for b in range(2):
                    j = g * 2 + b
                    pltpu.make_async_copy(
                        table.at[sidx.at[j]], rows.at[b], sems.at[b]
                    ).wait()
                    pltpu.sync_copy(rows.at[b], acc.at[didx.at[j]], add=True)

                    @pl.when(j + 2 < KE2)
                    def _next():
                        pltpu.async_copy(
                            table.at[sidx.at[j + 2]], rows.at[b], sems.at[b]
                        )

        plsc.subcore_barrier()
        pltpu.sync_copy(acc.at[pl.ds(r0, rpt)], out.at[c, pl.ds(r0, rpt)])

    return pl.kernel(
        body,
        out_type=jax.ShapeDtypeStruct((NC, NP, D), jnp.float32),
        mesh=_mesh(),
        scratch_types=[
            pltpu.VMEM_SHARED((NP, D), jnp.float32),
            pltpu.VMEM((KE2, BE), jnp.int32),
            pltpu.VMEM((KE2, BE), jnp.int32),
            pltpu.VMEM((2, BE, D), jnp.float32),
            pltpu.SemaphoreType.DMA((2,)),
        ],
    )


def _dinv_block(degp_ref):
    deg = degp_ref[0, :, 0:1] + degp_ref[1, :, 0:1]
    return lax.rsqrt(jnp.maximum(deg, 1.0))


def _layer0_body(x_ref, degp_ref, w_ref, out_ref):
    out_ref[...] = _dinv_block(degp_ref) * jnp.dot(
        x_ref[...], w_ref[...], preferred_element_type=jnp.float32
    )


def _mid_body(p_ref, degp_ref, b_ref, w_ref, out_ref):
    dinv = _dinv_block(degp_ref)
    xin = jnp.maximum(dinv * (p_ref[0] + p_ref[1]) + b_ref[...], 0.0)
    out_ref[...] = dinv * jnp.dot(
        xin, w_ref[...], preferred_element_type=jnp.float32
    )


def _final_body(p_ref, degp_ref, b_ref, out_ref):
    out_ref[...] = _dinv_block(degp_ref) * (p_ref[0] + p_ref[1]) + b_ref[...]


def _nodes_spec(D):
    return pl.BlockSpec((BN, D), lambda i: (i, 0))


def _pair_spec(D):
    return pl.BlockSpec((2, BN, D), lambda i: (0, i, 0))


def _full_spec(shape):
    return pl.BlockSpec(shape, lambda i: tuple(0 for _ in shape))


@functools.lru_cache(maxsize=None)
def _layer0_call(NP, D):
    return pl.pallas_call(
        _layer0_body,
        grid=(NP // BN,),
        in_specs=[_nodes_spec(D), _pair_spec(DW), _full_spec((D, D))],
        out_specs=_nodes_spec(D),
        out_shape=jax.ShapeDtypeStruct((NP, D), jnp.float32),
    )


@functools.lru_cache(maxsize=None)
def _mid_call(NP, D):
    return pl.pallas_call(
        _mid_body,
        grid=(NP // BN,),
        in_specs=[_pair_spec(D), _pair_spec(DW), _full_spec((1, D)),
                  _full_spec((D, D))],
        out_specs=_nodes_spec(D),
        out_shape=jax.ShapeDtypeStruct((NP, D), jnp.float32),
    )


@functools.lru_cache(maxsize=None)
def _final_call(NP, D):
    return pl.pallas_call(
        _final_body,
        grid=(NP // BN,),
        in_specs=[_pair_spec(D), _pair_spec(DW), _full_spec((1, D))],
        out_specs=_nodes_spec(D),
        out_shape=jax.ShapeDtypeStruct((NP, D), jnp.float32),
    )


def kernel(x, edge_index, W0, b0, W1, b1, W2, b2):
    N, D = x.shape
    E = edge_index.shape[1]

    NP = -(-(N + 1) // BN) * BN
    KE = 16 * (-(-E // (NW * BE * 16)))
    EP = NW * KE * BE

    pad_idx = N + (jnp.arange(EP - E, dtype=jnp.int32) % (NP - N))
    src = jnp.concatenate([edge_index[0], pad_idx]).reshape(NC, NS, KE, BE)
    dst = jnp.concatenate([edge_index[1], pad_idx]).reshape(NC, NS, KE, BE)
    x_pad = jnp.zeros((NP, D), jnp.float32).at[:N].set(x)
    zrows = jnp.zeros((NP // NS, D), jnp.float32)
    zdeg = jnp.zeros((NP // NS, DW), jnp.float32)
    ones_h = jnp.ones((BE, DW), jnp.float32)

    degp = _degree_kernel(NP, KE)(dst, zdeg, ones_h)

    h = _layer0_call(NP, D)(x_pad, degp, W0)
    p = _propagate_kernel(NP, KE, D)(h, src, dst, zrows)

    h = _mid_call(NP, D)(p, degp, b0.reshape(1, D), W1)
    p = _propagate_kernel(NP, KE, D)(h, src, dst, zrows)

    h = _mid_call(NP, D)(p, degp, b1.reshape(1, D), W2)
    p = _propagate_kernel(NP, KE, D)(h, src, dst, zrows)

    return _final_call(NP, D)(p, degp, b2.reshape(1, D))[:N]


# speedup vs baseline: 1.0181x; 1.0181x over previous
import functools

import jax
import jax.numpy as jnp
from jax import lax
from jax.experimental import pallas as pl
from jax.experimental.pallas import tpu as pltpu
from jax.experimental.pallas import tpu_sc as plsc

NC = 2
NS = 16
NW = NC * NS
BE = 128
BN = 1024


def _mesh():
    return plsc.VectorSubcoreMesh(core_axis_name="c", subcore_axis_name="s")


DW = 16


@functools.lru_cache(maxsize=None)
def _degree_kernel(NP, KE):
    rpt = NP // NS

    def body(dst3, zrows, ones_h, out, acc, didx, ones_v):
        c = lax.axis_index("c")
        s = lax.axis_index("s")
        r0 = s * rpt
        pltpu.sync_copy(dst3.at[c, s], didx)
        pltpu.sync_copy(zrows, acc.at[pl.ds(r0, rpt)])
        pltpu.sync_copy(ones_h, ones_v)
        plsc.subcore_barrier()

        @pl.loop(0, KE)
        def _scat(j):
            pltpu.sync_copy(ones_v, acc.at[didx.at[j]], add=True)

        plsc.subcore_barrier()
        pltpu.sync_copy(acc.at[pl.ds(r0, rpt)], out.at[c, pl.ds(r0, rpt)])

    return pl.kernel(
        body,
        out_type=jax.ShapeDtypeStruct((NC, NP, DW), jnp.float32),
        mesh=_mesh(),
        scratch_types=[
            pltpu.VMEM_SHARED((NP, DW), jnp.float32),
            pltpu.VMEM((KE, BE), jnp.int32),
            pltpu.VMEM((BE, DW), jnp.float32),
        ],
        compiler_params=pltpu.CompilerParams(use_tc_tiling_on_sc=False),
    )


@functools.lru_cache(maxsize=None)
def _propagate_kernel(NP, KE, D):
    rpt = NP // NS

    KE2 = KE // 2

    def body(table, src3, dst3, zrows, out, acc, sidx, didx, rows, sems, zsem):
        c = lax.axis_index("c")
        s = lax.axis_index("s")
        r0 = s * rpt
        zdesc = pltpu.async_copy(zrows, acc.at[pl.ds(r0, rpt)], zsem)

        for p in range(2):
            pltpu.sync_copy(src3.at[c, s, pl.ds(p * KE2, KE2)], sidx)
            pltpu.sync_copy(dst3.at[c, s, pl.ds(p * KE2, KE2)], didx)
            pltpu.async_copy(table.at[sidx.at[0]], rows.at[0], sems.at[0])
            pltpu.async_copy(table.at[sidx.at[1]], rows.at[1], sems.at[1])
            if p == 0:
                zdesc.wait()
                plsc.subcore_barrier()

            @pl.loop(0, KE2 // 2)
            def _go(g):
                for b in range(2):
                    j = g * 2 + b
                    pltpu.make_async_copy(
                        table.at[sidx.at[j]], rows.at[b], sems.at[b]
                    ).wait()
                    pltpu.sync_copy(rows.at[b], acc.at[didx.at[j]], add=True)

                    @pl.when(j + 2 < KE2)
                    def _next():
                        pltpu.async_copy(
                            table.at[sidx.at[j + 2]], rows.at[b], sems.at[b]
                        )

        plsc.subcore_barrier()
        pltpu.sync_copy(acc.at[pl.ds(r0, rpt)], out.at[c, pl.ds(r0, rpt)])

    return pl.kernel(
        body,
        out_type=jax.ShapeDtypeStruct((NC, NP, D), jnp.float32),
        mesh=_mesh(),
        scratch_types=[
            pltpu.VMEM_SHARED((NP, D), jnp.float32),
            pltpu.VMEM((KE2, BE), jnp.int32),
            pltpu.VMEM((KE2, BE), jnp.int32),
            pltpu.VMEM((2, BE, D), jnp.float32),
            pltpu.SemaphoreType.DMA((2,)),
            pltpu.SemaphoreType.DMA,
        ],
    )


def _dinv_block(degp_ref):
    deg = degp_ref[0, :, 0:1] + degp_ref[1, :, 0:1]
    return lax.rsqrt(jnp.maximum(deg, 1.0))


def _layer0_body(x_ref, degp_ref, w_ref, out_ref):
    out_ref[...] = _dinv_block(degp_ref) * jnp.dot(
        x_ref[...], w_ref[...], preferred_element_type=jnp.float32
    )


def _mid_body(p_ref, degp_ref, b_ref, w_ref, out_ref):
    dinv = _dinv_block(degp_ref)
    xin = jnp.maximum(dinv * (p_ref[0] + p_ref[1]) + b_ref[...], 0.0)
    out_ref[...] = dinv * jnp.dot(
        xin, w_ref[...], preferred_element_type=jnp.float32
    )


def _final_body(p_ref, degp_ref, b_ref, out_ref):
    out_ref[...] = _dinv_block(degp_ref) * (p_ref[0] + p_ref[1]) + b_ref[...]


def _nodes_spec(D):
    return pl.BlockSpec((BN, D), lambda i: (i, 0))


def _pair_spec(D):
    return pl.BlockSpec((2, BN, D), lambda i: (0, i, 0))


def _full_spec(shape):
    return pl.BlockSpec(shape, lambda i: tuple(0 for _ in shape))


@functools.lru_cache(maxsize=None)
def _layer0_call(NP, D):
    return pl.pallas_call(
        _layer0_body,
        grid=(NP // BN,),
        in_specs=[_nodes_spec(D), _pair_spec(DW), _full_spec((D, D))],
        out_specs=_nodes_spec(D),
        out_shape=jax.ShapeDtypeStruct((NP, D), jnp.float32),
    )


@functools.lru_cache(maxsize=None)
def _mid_call(NP, D):
    return pl.pallas_call(
        _mid_body,
        grid=(NP // BN,),
        in_specs=[_pair_spec(D), _pair_spec(DW), _full_spec((1, D)),
                  _full_spec((D, D))],
        out_specs=_nodes_spec(D),
        out_shape=jax.ShapeDtypeStruct((NP, D), jnp.float32),
    )


@functools.lru_cache(maxsize=None)
def _final_call(NP, D):
    return pl.pallas_call(
        _final_body,
        grid=(NP // BN,),
        in_specs=[_pair_spec(D), _pair_spec(DW), _full_spec((1, D))],
        out_specs=_nodes_spec(D),
        out_shape=jax.ShapeDtypeStruct((NP, D), jnp.float32),
    )


def kernel(x, edge_index, W0, b0, W1, b1, W2, b2):
    N, D = x.shape
    E = edge_index.shape[1]

    NP = -(-(N + 1) // BN) * BN
    KE = 16 * (-(-E // (NW * BE * 16)))
    EP = NW * KE * BE

    pad_idx = N + (jnp.arange(EP - E, dtype=jnp.int32) % (NP - N))
    src = jnp.concatenate([edge_index[0], pad_idx]).reshape(NC, NS, KE, BE)
    dst = jnp.concatenate([edge_index[1], pad_idx]).reshape(NC, NS, KE, BE)
    x_pad = jnp.zeros((NP, D), jnp.float32).at[:N].set(x)
    zrows = jnp.zeros((NP // NS, D), jnp.float32)
    zdeg = jnp.zeros((NP // NS, DW), jnp.float32)
    ones_h = jnp.ones((BE, DW), jnp.float32)

    degp = _degree_kernel(NP, KE)(dst, zdeg, ones_h)

    h = _layer0_call(NP, D)(x_pad, degp, W0)
    p = _propagate_kernel(NP, KE, D)(h, src, dst, zrows)

    h = _mid_call(NP, D)(p, degp, b0.reshape(1, D), W1)
    p = _propagate_kernel(NP, KE, D)(h, src, dst, zrows)

    h = _mid_call(NP, D)(p, degp, b1.reshape(1, D), W2)
    p = _propagate_kernel(NP, KE, D)(h, src, dst, zrows)

    return _final_call(NP, D)(p, degp, b2.reshape(1, D))[:N]
